# baseline (speedup 1.0000x reference)
import jax
import jax.numpy as jnp
import numpy as np
from jax.experimental import pallas as pl
from jax.experimental.pallas import tpu as pltpu

KE_KCAL = 332.0716
R_ON = 4.0

N = 2048
BI = 512
BJ = 512
_NB = N // BI
_BIS = np.array([i for i in range(_NB) for j in range(i, _NB)], dtype=np.int32)
_BJS = np.array([j for i in range(_NB) for j in range(i, _NB)], dtype=np.int32)
_T = len(_BIS)


def _tile_energy(colb, rowb, triangle):
    dx = colb[:, 0:1] - rowb[0:1, :]
    dy = colb[:, 1:2] - rowb[1:2, :]
    dz = colb[:, 2:3] - rowb[2:3, :]
    r2 = dx * dx + dy * dy + dz * dz
    qq = colb[:, 3:4] * rowb[3:4, :]
    if triangle:
        il = jax.lax.broadcasted_iota(jnp.int32, (BI, BJ), 0)
        jl = jax.lax.broadcasted_iota(jnp.int32, (BI, BJ), 1)
        qq = jnp.where(jl > il, qq, 0.0)

    r2c = jnp.maximum(r2, 1e-30)
    rinv = jax.lax.rsqrt(r2c)
    r = r2c * rinv
    a = jnp.clip(r - R_ON, 0.0, 1.0)
    v = (a - 0.5) / (a - a * a)
    t = jnp.tanh(v)
    g = jax.lax.rsqrt(r2c + 1.0)
    e = qq * ((g + rinv) + t * (rinv - g))
    return jnp.sum(e, axis=(0, 1), keepdims=True)


def _tile_kernel(bis_ref, bjs_ref, col_ref, rowsrc_ref, out_ref):
    t = pl.program_id(0)
    bi = bis_ref[t]
    bj = bjs_ref[t]

    colb = col_ref[...]
    rowb = jnp.transpose(rowsrc_ref[...])

    @pl.when(t == 0)
    def _():
        out_ref[...] = jnp.zeros_like(out_ref)

    @pl.when(bi == bj)
    def _():
        out_ref[...] += _tile_energy(colb, rowb, triangle=True)

    @pl.when(bi != bj)
    def _():
        out_ref[...] += _tile_energy(colb, rowb, triangle=False)


@jax.jit
def kernel(q, xyz):
    n = xyz.shape[0]
    qs = q[:, None] * np.float32(np.sqrt(KE_KCAL / 2.0))
    packed = jnp.concatenate([xyz, qs], axis=1)

    grid_spec = pltpu.PrefetchScalarGridSpec(
        num_scalar_prefetch=2,
        grid=(_T,),
        in_specs=[
            pl.BlockSpec((BI, 4), lambda t, bis, bjs: (bis[t], 0)),
            pl.BlockSpec((BJ, 4), lambda t, bis, bjs: (bjs[t], 0)),
        ],
        out_specs=pl.BlockSpec((1, 1), lambda t, bis, bjs: (0, 0)),
    )
    out = pl.pallas_call(
        _tile_kernel,
        grid_spec=grid_spec,
        out_shape=jax.ShapeDtypeStruct((1, 1), jnp.float32),
        compiler_params=pltpu.CompilerParams(
            dimension_semantics=("arbitrary",),
        ),
    )(jnp.asarray(_BIS), jnp.asarray(_BJS), packed, packed)
    return out[0, 0]


# speedup vs baseline: 1.0241x; 1.0241x over previous
import jax
import jax.numpy as jnp
import numpy as np
from jax.experimental import pallas as pl
from jax.experimental.pallas import tpu as pltpu

KE_KCAL = 332.0716
R_ON = 4.0

N = 2048
BI = 512
BJ = 512
_NB = N // BI
_BIS = np.array([i for i in range(_NB) for j in range(i, _NB)], dtype=np.int32)
_BJS = np.array([j for i in range(_NB) for j in range(i, _NB)], dtype=np.int32)
_T = len(_BIS)


def _tile_energy(colb, rowb, triangle):
    dx = colb[:, 0:1] - rowb[0:1, :]
    dy = colb[:, 1:2] - rowb[1:2, :]
    dz = colb[:, 2:3] - rowb[2:3, :]
    r2 = dx * dx + dy * dy + dz * dz
    qq = colb[:, 3:4] * rowb[3:4, :]
    if triangle:
        il = jax.lax.broadcasted_iota(jnp.int32, (BI, BJ), 0)
        jl = jax.lax.broadcasted_iota(jnp.int32, (BI, BJ), 1)
        qq = jnp.where(jl > il, qq, 0.0)

    r2c = jnp.maximum(r2, 1e-30)
    rinv = jax.lax.rsqrt(r2c)
    r = r2c * rinv
    a = jnp.clip(r - R_ON, 0.0, 1.0)
    v = (a - 0.5) / (a - a * a)
    fs = 0.5 - 0.5 * jnp.tanh(v)
    g = jax.lax.rsqrt(r2c + 1.0)
    e = qq * (fs * (g - rinv) + rinv)
    return jnp.sum(e, axis=(0, 1), keepdims=True)


def _tile_kernel(bis_ref, bjs_ref, col_ref, rowsrc_ref, out_ref):
    t = pl.program_id(0)
    bi = bis_ref[t]
    bj = bjs_ref[t]

    colb = col_ref[...]
    rowb = jnp.transpose(rowsrc_ref[...])

    @pl.when(t == 0)
    def _():
        out_ref[...] = jnp.zeros_like(out_ref)

    @pl.when(bi == bj)
    def _():
        out_ref[...] += KE_KCAL * _tile_energy(colb, rowb, triangle=True)

    @pl.when(bi != bj)
    def _():
        out_ref[...] += KE_KCAL * _tile_energy(colb, rowb, triangle=False)


@jax.jit
def kernel(q, xyz):
    n = xyz.shape[0]
    packed = jnp.concatenate([xyz, q[:, None]], axis=1)

    grid_spec = pltpu.PrefetchScalarGridSpec(
        num_scalar_prefetch=2,
        grid=(_T,),
        in_specs=[
            pl.BlockSpec((BI, 4), lambda t, bis, bjs: (bis[t], 0)),
            pl.BlockSpec((BJ, 4), lambda t, bis, bjs: (bjs[t], 0)),
        ],
        out_specs=pl.BlockSpec((1, 1), lambda t, bis, bjs: (0, 0)),
    )
    out = pl.pallas_call(
        _tile_kernel,
        grid_spec=grid_spec,
        out_shape=jax.ShapeDtypeStruct((1, 1), jnp.float32),
        compiler_params=pltpu.CompilerParams(
            dimension_semantics=("arbitrary",),
        ),
    )(jnp.asarray(_BIS), jnp.asarray(_BJS), packed, packed)
    return out[0, 0]


# speedup vs baseline: 1.0386x; 1.0142x over previous
import jax
import jax.numpy as jnp
import numpy as np
from jax.experimental import pallas as pl
from jax.experimental.pallas import tpu as pltpu

KE_KCAL = 332.0716
R_ON = 4.0

N = 2048
BI = 512
BJ = 512
_NB = N // BI
_BIS = np.array([i for i in range(_NB) for j in range(i, _NB)], dtype=np.int32)
_BJS = np.array([j for i in range(_NB) for j in range(i, _NB)], dtype=np.int32)
_T = len(_BIS)


def _tile_energy(colb, rowb, triangle):
    dx = colb[:, 0:1] - rowb[0:1, :]
    dy = colb[:, 1:2] - rowb[1:2, :]
    dz = colb[:, 2:3] - rowb[2:3, :]
    r2 = dx * dx + dy * dy + dz * dz
    qq = colb[:, 3:4] * rowb[3:4, :]
    if triangle:
        il = jax.lax.broadcasted_iota(jnp.int32, (BI, BJ), 0)
        jl = jax.lax.broadcasted_iota(jnp.int32, (BI, BJ), 1)
        qq = jnp.where(jl > il, qq, 0.0)

    r2c = jnp.maximum(r2, 1e-30)
    rinv = jax.lax.rsqrt(r2c)
    r = r2c * rinv
    a = jnp.clip(r - R_ON, 0.0, 1.0)
    v = (a - 0.5) / (a - a * a)
    fs = 0.5 - 0.5 * jnp.tanh(v)
    g = jax.lax.rsqrt(r2c + 1.0)
    e = qq * (fs * (g - rinv) + rinv)
    return jnp.sum(e, axis=(0, 1), keepdims=True)


def _tile_kernel(bis_ref, bjs_ref, col_ref, row_ref, out_ref):
    t = pl.program_id(0)
    bi = bis_ref[t]
    bj = bjs_ref[t]

    colb = col_ref[...]
    rowb = row_ref[...]

    @pl.when(t == 0)
    def _():
        out_ref[...] = jnp.zeros_like(out_ref)

    @pl.when(bi == bj)
    def _():
        out_ref[...] += KE_KCAL * _tile_energy(colb, rowb, triangle=True)

    @pl.when(bi != bj)
    def _():
        out_ref[...] += KE_KCAL * _tile_energy(colb, rowb, triangle=False)


@jax.jit
def kernel(q, xyz):
    n = xyz.shape[0]
    packed = jnp.concatenate([xyz, q[:, None]], axis=1)
    rows = packed.T

    grid_spec = pltpu.PrefetchScalarGridSpec(
        num_scalar_prefetch=2,
        grid=(_T,),
        in_specs=[
            pl.BlockSpec((BI, 4), lambda t, bis, bjs: (bis[t], 0)),
            pl.BlockSpec((4, BJ), lambda t, bis, bjs: (0, bjs[t])),
        ],
        out_specs=pl.BlockSpec((1, 1), lambda t, bis, bjs: (0, 0)),
    )
    out = pl.pallas_call(
        _tile_kernel,
        grid_spec=grid_spec,
        out_shape=jax.ShapeDtypeStruct((1, 1), jnp.float32),
        compiler_params=pltpu.CompilerParams(
            dimension_semantics=("arbitrary",),
        ),
    )(jnp.asarray(_BIS), jnp.asarray(_BJS), packed, rows)
    return out[0, 0]
